# baseline (speedup 1.0000x reference)
import jax
import jax.numpy as jnp
import numpy as np
from jax import lax
from jax.experimental import pallas as pl
from jax.experimental.pallas import tpu as pltpu
from jax.experimental.pallas import tpu_sc as plsc

VOCAB = 1000000
D_MODEL = 768
R = 128
B = 4096
L = 200
BL = B * L

NC = 2
NS = 16
NW = NC * NS

CHUNK = 256
N_SUB = CHUNK // 128
NBUF = 2

SLAB_STEPS = (2, 6, 12, 22, 40, 18)
SLAB_ROWS = tuple(s * CHUNK * NW for s in SLAB_STEPS)
SLABS = len(SLAB_STEPS)


def _make_gather_body(steps, per_w, slab_row0):
    def _gather_body(
        ids_hbm, table_hbm, out_hbm, idx_v, rows_v, pk_v, gsem, osem
    ):
        c = lax.axis_index("c")
        s = lax.axis_index("s")
        wid = s * NC + c
        base_row = slab_row0 + wid * (per_w // 128)

        def fire(g, b):
            row = base_row + g * N_SUB
            pltpu.sync_copy(ids_hbm.at[pl.ds(row, N_SUB)], idx_v.at[b])
            for j in range(N_SUB):
                pltpu.async_copy(
                    table_hbm.at[idx_v.at[b].at[j]],
                    rows_v.at[b].at[pl.ds(j * 128, 128)],
                    gsem,
                )

        def drain_pack_store(g, b):
            for j in range(N_SUB):
                pltpu.make_async_copy(
                    table_hbm.at[idx_v.at[b].at[j]],
                    rows_v.at[b].at[pl.ds(j * 128, 128)],
                    gsem,
                ).wait()

            def pack_row(r):
                for grp in range(R // 32):
                    a = rows_v[b, r, pl.ds(grp * 32, 16)]
                    bb = rows_v[b, r, pl.ds(grp * 32 + 16, 16)]
                    pk_v[b, r, pl.ds(grp * 32, 32)] = plsc.pack(
                        a, bb, format=plsc.PackFormat.INTERLEAVED
                    )

            pl.loop(0, CHUNK)(pack_row)
            pltpu.async_copy(
                pk_v.at[b],
                out_hbm.at[pl.ds(wid * per_w + g * CHUNK, CHUNK)],
                osem,
            )

        def wait_store(g, b):
            pltpu.make_async_copy(
                pk_v.at[b],
                out_hbm.at[pl.ds(wid * per_w + g * CHUNK, CHUNK)],
                osem,
            ).wait()

        fire(0, 0)

        def step(g):
            @pl.when(g >= 2)
            def _():
                wait_store(g - 1, 1)

            fire(g + 1, 1)
            drain_pack_store(g, 0)

            @pl.when(g + 2 < steps)
            def _():
                wait_store(g, 0)
                fire(g + 2, 0)

            drain_pack_store(g + 1, 1)

        pl.loop(0, steps, step=NBUF)(step)
        wait_store(steps - 2, 0)
        wait_store(steps - 1, 1)

    return _gather_body


def _sc_gather_slab(ids2d, table, steps, slab_row0):
    rows = steps * CHUNK * NW
    kern = pl.kernel(
        _make_gather_body(steps, steps * CHUNK, slab_row0),
        out_type=jax.ShapeDtypeStruct((rows, R), jnp.bfloat16),
        mesh=plsc.VectorSubcoreMesh(core_axis_name="c", subcore_axis_name="s"),
        compiler_params=pltpu.CompilerParams(needs_layout_passes=False),
        scratch_types=[
            pltpu.VMEM((NBUF, N_SUB, 128), jnp.int32),
            pltpu.VMEM((NBUF, CHUNK, R), jnp.float32),
            pltpu.VMEM((NBUF, CHUNK, R), jnp.bfloat16),
            pltpu.SemaphoreType.DMA,
            pltpu.SemaphoreType.DMA,
        ],
    )
    return kern(ids2d, table)


RB = 4096


def _proj_first_body(x_ref, w_ref, o_ref):
    o_ref[...] = jnp.dot(
        x_ref[...], w_ref[...], preferred_element_type=jnp.float32
    )


def _proj_next_body(prev_ref, x_ref, w_ref, o_ref):
    del prev_ref
    o_ref[...] = jnp.dot(
        x_ref[...], w_ref[...], preferred_element_type=jnp.float32
    )


def _tc_project_slab(block_off, n_blocks, prev_out, x, w_t):
    out_spec = pl.BlockSpec(
        (RB, D_MODEL), lambda i, off=block_off: (off + i, 0)
    )
    x_spec = pl.BlockSpec((RB, R), lambda i: (i, 0))
    w_spec = pl.BlockSpec((R, D_MODEL), lambda i: (0, 0))
    params = pltpu.CompilerParams(dimension_semantics=("arbitrary",))
    if prev_out is None:
        return pl.pallas_call(
            _proj_first_body,
            grid=(n_blocks,),
            in_specs=[x_spec, w_spec],
            out_specs=out_spec,
            out_shape=jax.ShapeDtypeStruct((BL, D_MODEL), jnp.float32),
            compiler_params=params,
        )(x, w_t)
    return pl.pallas_call(
        _proj_next_body,
        grid=(n_blocks,),
        in_specs=[
            pl.BlockSpec(memory_space=pl.ANY),
            x_spec,
            w_spec,
        ],
        out_specs=out_spec,
        out_shape=jax.ShapeDtypeStruct((BL, D_MODEL), jnp.float32),
        input_output_aliases={0: 0},
        compiler_params=params,
    )(prev_out, x, w_t)


_p = np.arange(R)
_PACK_PERM = 32 * (_p // 32) + 16 * (_p % 2) + (_p % 32) // 2


@jax.jit
def _run(ids2d, table, w_t_perm):
    gathered = []
    row0 = 0
    for s in range(SLABS):
        gathered.append(
            _sc_gather_slab(ids2d, table, SLAB_STEPS[s], row0 // 128)
        )
        row0 += SLAB_ROWS[s]
    out = None
    block_off = 0
    for s in range(SLABS):
        n_blocks = SLAB_ROWS[s] // RB
        out = _tc_project_slab(block_off, n_blocks, out, gathered[s], w_t_perm)
        block_off += n_blocks
    return out


def kernel(input_ids, embed_weight, proj_weight):
    ids2d = input_ids.reshape(BL // 128, 128).astype(jnp.int32)
    w_t_perm = proj_weight.T[_PACK_PERM].astype(jnp.bfloat16)
    out = _run(ids2d, embed_weight, w_t_perm)
    return out.reshape(B, L, D_MODEL)


# speedup vs baseline: 1.0158x; 1.0158x over previous
import jax
import jax.numpy as jnp
import numpy as np
from jax import lax
from jax.experimental import pallas as pl
from jax.experimental.pallas import tpu as pltpu
from jax.experimental.pallas import tpu_sc as plsc

VOCAB = 1000000
D_MODEL = 768
R = 128
B = 4096
L = 200
BL = B * L

NC = 2
NS = 16
NW = NC * NS

CHUNK = 256
N_SUB = CHUNK // 128
NBUF = 2

SLAB_STEPS = (4, 6, 12, 22, 40, 16)
SLAB_ROWS = tuple(s * CHUNK * NW for s in SLAB_STEPS)
SLABS = len(SLAB_STEPS)


def _make_gather_body(steps, per_w, slab_row0):
    def _gather_body(
        ids_hbm, table_hbm, out_hbm, idx_v, rows_v, pk_v, gsem, osem
    ):
        c = lax.axis_index("c")
        s = lax.axis_index("s")
        wid = s * NC + c
        base_row = slab_row0 + wid * (per_w // 128)

        def fire(g, b):
            row = base_row + g * N_SUB
            pltpu.sync_copy(ids_hbm.at[pl.ds(row, N_SUB)], idx_v.at[b])
            for j in range(N_SUB):
                pltpu.async_copy(
                    table_hbm.at[idx_v.at[b].at[j]],
                    rows_v.at[b].at[pl.ds(j * 128, 128)],
                    gsem,
                )

        def drain_pack_store(g, b):
            for j in range(N_SUB):
                pltpu.make_async_copy(
                    table_hbm.at[idx_v.at[b].at[j]],
                    rows_v.at[b].at[pl.ds(j * 128, 128)],
                    gsem,
                ).wait()

            def pack_row(r):
                for grp in range(R // 32):
                    a = rows_v[b, r, pl.ds(grp * 32, 16)]
                    bb = rows_v[b, r, pl.ds(grp * 32 + 16, 16)]
                    pk_v[b, r, pl.ds(grp * 32, 32)] = plsc.pack(
                        a, bb, format=plsc.PackFormat.INTERLEAVED
                    )

            pl.loop(0, CHUNK)(pack_row)
            pltpu.async_copy(
                pk_v.at[b],
                out_hbm.at[pl.ds(wid * per_w + g * CHUNK, CHUNK)],
                osem,
            )

        def wait_store(g, b):
            pltpu.make_async_copy(
                pk_v.at[b],
                out_hbm.at[pl.ds(wid * per_w + g * CHUNK, CHUNK)],
                osem,
            ).wait()

        fire(0, 0)

        def step(g):
            @pl.when(g >= 2)
            def _():
                wait_store(g - 1, 1)

            fire(g + 1, 1)
            drain_pack_store(g, 0)

            @pl.when(g + 2 < steps)
            def _():
                wait_store(g, 0)
                fire(g + 2, 0)

            drain_pack_store(g + 1, 1)

        pl.loop(0, steps, step=NBUF)(step)
        wait_store(steps - 2, 0)
        wait_store(steps - 1, 1)

    return _gather_body


def _sc_gather_slab(ids2d, table, steps, slab_row0):
    rows = steps * CHUNK * NW
    kern = pl.kernel(
        _make_gather_body(steps, steps * CHUNK, slab_row0),
        out_type=jax.ShapeDtypeStruct((rows, R), jnp.bfloat16),
        mesh=plsc.VectorSubcoreMesh(core_axis_name="c", subcore_axis_name="s"),
        compiler_params=pltpu.CompilerParams(needs_layout_passes=False),
        scratch_types=[
            pltpu.VMEM((NBUF, N_SUB, 128), jnp.int32),
            pltpu.VMEM((NBUF, CHUNK, R), jnp.float32),
            pltpu.VMEM((NBUF, CHUNK, R), jnp.bfloat16),
            pltpu.SemaphoreType.DMA,
            pltpu.SemaphoreType.DMA,
        ],
    )
    return kern(ids2d, table)


RB = 8192


def _proj_first_body(x_ref, w_ref, o_ref):
    o_ref[...] = jnp.dot(
        x_ref[...], w_ref[...], preferred_element_type=jnp.float32
    )


def _proj_next_body(prev_ref, x_ref, w_ref, o_ref):
    del prev_ref
    o_ref[...] = jnp.dot(
        x_ref[...], w_ref[...], preferred_element_type=jnp.float32
    )


def _tc_project_slab(block_off, n_blocks, prev_out, x, w_t):
    out_spec = pl.BlockSpec(
        (RB, D_MODEL), lambda i, off=block_off: (off + i, 0)
    )
    x_spec = pl.BlockSpec((RB, R), lambda i: (i, 0))
    w_spec = pl.BlockSpec((R, D_MODEL), lambda i: (0, 0))
    params = pltpu.CompilerParams(dimension_semantics=("arbitrary",))
    if prev_out is None:
        return pl.pallas_call(
            _proj_first_body,
            grid=(n_blocks,),
            in_specs=[x_spec, w_spec],
            out_specs=out_spec,
            out_shape=jax.ShapeDtypeStruct((BL, D_MODEL), jnp.float32),
            compiler_params=params,
        )(x, w_t)
    return pl.pallas_call(
        _proj_next_body,
        grid=(n_blocks,),
        in_specs=[
            pl.BlockSpec(memory_space=pl.ANY),
            x_spec,
            w_spec,
        ],
        out_specs=out_spec,
        out_shape=jax.ShapeDtypeStruct((BL, D_MODEL), jnp.float32),
        input_output_aliases={0: 0},
        compiler_params=params,
    )(prev_out, x, w_t)


_p = np.arange(R)
_PACK_PERM = 32 * (_p // 32) + 16 * (_p % 2) + (_p % 32) // 2


@jax.jit
def _run(ids2d, table, w_t_perm):
    gathered = []
    row0 = 0
    for s in range(SLABS):
        gathered.append(
            _sc_gather_slab(ids2d, table, SLAB_STEPS[s], row0 // 128)
        )
        row0 += SLAB_ROWS[s]
    out = None
    block_off = 0
    for s in range(SLABS):
        n_blocks = SLAB_ROWS[s] // RB
        out = _tc_project_slab(block_off, n_blocks, out, gathered[s], w_t_perm)
        block_off += n_blocks
    return out


def kernel(input_ids, embed_weight, proj_weight):
    ids2d = input_ids.reshape(BL // 128, 128).astype(jnp.int32)
    w_t_perm = proj_weight.T[_PACK_PERM].astype(jnp.bfloat16)
    out = _run(ids2d, embed_weight, w_t_perm)
    return out.reshape(B, L, D_MODEL)
